# Initial kernel scaffold; baseline (speedup 1.0000x reference)
import jax
import jax.numpy as jnp
from jax.experimental import pallas as pl


def kernel(indices, table, dummy):
    raise NotImplementedError("write your pallas kernel here")


# speedup vs baseline: 1.5078x; 1.5078x over previous
import functools

import jax
import jax.numpy as jnp
from jax import lax
from jax.experimental import pallas as pl
from jax.experimental.pallas import tpu as pltpu
from jax.experimental.pallas import tpu_sc as plsc


def _make_sc_gather(V, D, N):
    info = plsc.get_sparse_core_info()
    NW = info.num_cores * info.num_subcores
    assert N % NW == 0
    n_per_w = N // NW
    CH = 832
    assert n_per_w % CH == 0
    n_ch = n_per_w // CH
    mesh = plsc.VectorSubcoreMesh(core_axis_name="c", subcore_axis_name="s")

    @functools.partial(
        pl.kernel,
        mesh=mesh,
        out_type=jax.ShapeDtypeStruct((N, D), jnp.float32),
        compiler_params=pltpu.CompilerParams(use_tc_tiling_on_sc=False),
        scratch_types=[
            pltpu.VMEM((n_per_w,), jnp.int32),
            pltpu.VMEM((2, CH, D), jnp.float32),
            pltpu.SemaphoreType.DMA,
            pltpu.SemaphoreType.DMA,
            pltpu.SemaphoreType.DMA,
            pltpu.SemaphoreType.DMA,
        ],
    )
    def gather_kernel(idx_hbm, table_hbm, out_hbm, idx_v, rows_v,
                      gsem0, gsem1, osem0, osem1):
        gsem = (gsem0, gsem1)
        osem = (osem0, osem1)
        wid = lax.axis_index("s") * info.num_cores + lax.axis_index("c")
        base = wid * n_per_w
        pltpu.sync_copy(idx_hbm.at[pl.ds(base, n_per_w)], idx_v)

        def start_gather(i):
            b = i % 2
            return pltpu.async_copy(
                table_hbm.at[idx_v.at[pl.ds(i * CH, CH)]], rows_v.at[b],
                gsem[b])

        g_cur = start_gather(0)
        out_handles = [None, None]
        for i in range(n_ch):
            b = i % 2
            if i + 1 < n_ch:
                nb = (i + 1) % 2
                if out_handles[nb] is not None:
                    out_handles[nb].wait()
                g_next = start_gather(i + 1)
            g_cur.wait()
            out_handles[b] = pltpu.async_copy(
                rows_v.at[b], out_hbm.at[pl.ds(base + i * CH, CH)], osem[b])
            if i + 1 < n_ch:
                g_cur = g_next
        for h in out_handles:
            if h is not None:
                h.wait()

    return gather_kernel


def kernel(indices, table, dummy):
    B, F = indices.shape
    V, D = table.shape
    N = B * F
    idx_flat = indices.reshape(N)
    out = _make_sc_gather(V, D, N)(idx_flat, table)
    return out.reshape(B, F, D)
